# baseline (speedup 1.0000x reference)
import functools

import jax
import jax.numpy as jnp
from jax import lax
from jax.experimental import pallas as pl
from jax.experimental.pallas import tpu as pltpu
from jax.experimental.pallas import tpu_sc as plsc

_NC = 2
_NS = 16


def _edge_premix(x, w1a, w1b, b1, blk):
    n, d = x.shape

    def body(x_ref, wa_ref, wb_ref, b1_ref, a_ref, b_ref):
        xv = x_ref[...]
        a_ref[...] = jnp.dot(xv, wa_ref[...], preferred_element_type=jnp.float32)
        b_ref[...] = (
            jnp.dot(xv, wb_ref[...], preferred_element_type=jnp.float32)
            + b1_ref[...]
        )

    return pl.pallas_call(
        body,
        grid=(n // blk,),
        in_specs=[
            pl.BlockSpec((blk, d), lambda i: (i, 0)),
            pl.BlockSpec((d, d), lambda i: (0, 0)),
            pl.BlockSpec((d, d), lambda i: (0, 0)),
            pl.BlockSpec((1, d), lambda i: (0, 0)),
        ],
        out_specs=[
            pl.BlockSpec((blk, d), lambda i: (i, 0)),
            pl.BlockSpec((blk, d), lambda i: (i, 0)),
        ],
        out_shape=[
            jax.ShapeDtypeStruct((n, d), jnp.float32),
            jax.ShapeDtypeStruct((n, d), jnp.float32),
        ],
    )(x, w1a, w1b, b1)


def _sc_gather(a, b, src, dst, ch, nslot=4):
    n, d = a.shape
    e = src.shape[0]
    nw = _NC * _NS
    epw = e // nw
    nchunk = epw // ch
    assert nchunk >= nslot + 1 and (nchunk - 1) % nslot == 0
    mesh = plsc.VectorSubcoreMesh(
        core_axis_name="c", subcore_axis_name="s",
        num_cores=_NC, num_subcores=_NS,
    )

    @functools.partial(
        pl.kernel,
        mesh=mesh,
        out_type=(
            jax.ShapeDtypeStruct((e, d), jnp.float32),
            jax.ShapeDtypeStruct((e, d), jnp.float32),
        ),
        scratch_types=[pltpu.VMEM((epw,), jnp.int32)] * 2
        + [pltpu.VMEM((ch, d), jnp.float32)] * (2 * nslot)
        + [pltpu.SemaphoreType.DMA] * (2 * nslot),
    )
    def k(a_hbm, b_hbm, src_hbm, dst_hbm, ao_hbm, bo_hbm, src_v, dst_v,
          *rest):
        ra = rest[0:nslot]
        rb = rest[nslot:2 * nslot]
        sg = rest[2 * nslot:3 * nslot]
        sw = rest[3 * nslot:4 * nslot]
        wid = lax.axis_index("s") * _NC + lax.axis_index("c")
        base = wid * epw
        pltpu.sync_copy(src_hbm.at[pl.ds(base, epw)], src_v)
        pltpu.sync_copy(dst_hbm.at[pl.ds(base, epw)], dst_v)

        def fire_g(j, q):
            off = j * ch
            pltpu.async_copy(a_hbm.at[src_v.at[pl.ds(off, ch)]], ra[q], sg[q])
            pltpu.async_copy(b_hbm.at[dst_v.at[pl.ds(off, ch)]], rb[q], sg[q])

        def wait_g(q):
            pltpu.make_async_copy(a_hbm.at[pl.ds(0, ch)], ra[q], sg[q]).wait()
            pltpu.make_async_copy(b_hbm.at[pl.ds(0, ch)], rb[q], sg[q]).wait()

        def fire_w(j, q):
            off = base + j * ch
            pltpu.async_copy(ra[q], ao_hbm.at[pl.ds(off, ch)], sw[q])
            pltpu.async_copy(rb[q], bo_hbm.at[pl.ds(off, ch)], sw[q])

        def wait_w(q):
            pltpu.make_async_copy(ra[q], ao_hbm.at[pl.ds(0, ch)], sw[q]).wait()
            pltpu.make_async_copy(rb[q], bo_hbm.at[pl.ds(0, ch)], sw[q]).wait()

        for q in range(nslot):
            fire_g(q, q)

        def body(p, carry):
            for q in range(nslot):
                j = p * nslot + q
                wait_g(q)
                fire_w(j, q)

                @pl.when(j + nslot < nchunk)
                def _(q=q, j=j):
                    wait_w(q)
                    fire_g(j + nslot, q)

            return carry

        lax.fori_loop(0, (nchunk - 1) // nslot, body, 0)
        wait_g(0)
        fire_w(nchunk - 1, 0)
        for q in range(nslot):
            wait_w(q)

    return k(a, b, src, dst)


def _edge_mlp(ga, gb, w2, b2, blk):
    e, d = ga.shape

    def body(ga_ref, gb_ref, w2_ref, b2_ref, o_ref):
        h = jnp.maximum(ga_ref[...] + gb_ref[...], 0.0)
        m = (
            jnp.dot(h, w2_ref[...], preferred_element_type=jnp.float32)
            + b2_ref[...]
        )
        o_ref[...] = jnp.maximum(m, 0.0)

    return pl.pallas_call(
        body,
        grid=(e // blk,),
        in_specs=[
            pl.BlockSpec((blk, d), lambda i: (i, 0)),
            pl.BlockSpec((blk, d), lambda i: (i, 0)),
            pl.BlockSpec((d, d), lambda i: (0, 0)),
            pl.BlockSpec((1, d), lambda i: (0, 0)),
        ],
        out_specs=pl.BlockSpec((blk, d), lambda i: (i, 0)),
        out_shape=jax.ShapeDtypeStruct((e, d), jnp.float32),
    )(ga, gb, w2, b2)


def _sc_scatter(msg, dst4, n, ch, nslot=4):
    e, d = msg.shape
    nchunk = dst4.shape[2]
    assert nchunk >= nslot + 1 and (nchunk - 1) % nslot == 0
    nps = ((n + _NS * 128 - 1) // (_NS * 128)) * 128
    npad = _NS * nps
    mesh = plsc.VectorSubcoreMesh(
        core_axis_name="c", subcore_axis_name="s",
        num_cores=_NC, num_subcores=_NS,
    )

    @functools.partial(
        pl.kernel,
        mesh=mesh,
        out_type=jax.ShapeDtypeStruct((_NC, npad, d), jnp.float32),
        scratch_types=[pltpu.VMEM((nchunk, ch), jnp.int32)]
        + [pltpu.VMEM((ch, d), jnp.float32)] * nslot
        + [pltpu.VMEM_SHARED((npad, d), jnp.float32)]
        + [pltpu.SemaphoreType.DMA] * (2 * nslot),
    )
    def k(msg_hbm, dst_hbm, out_hbm, idx_v, *rest):
        m = rest[0:nslot]
        acc_sh = rest[nslot]
        sr = rest[nslot + 1:2 * nslot + 1]
        ss = rest[2 * nslot + 1:3 * nslot + 1]
        cid = lax.axis_index("c")
        sid = lax.axis_index("s")
        ebase = cid * (e // _NC) + sid * (e // (_NC * _NS))

        z = jnp.zeros((16,), jnp.float32)

        def zbody(r, carry):
            for q in range(d // 16):
                m[0][r, pl.ds(q * 16, 16)] = z
            return carry

        lax.fori_loop(0, ch, zbody, 0)
        for t in range(nps // ch):
            pltpu.sync_copy(m[0], acc_sh.at[pl.ds(sid * nps + t * ch, ch)])
        plsc.subcore_barrier()

        pltpu.sync_copy(dst_hbm.at[cid, sid], idx_v)

        def fire_r(j, q):
            pltpu.async_copy(msg_hbm.at[pl.ds(ebase + j * ch, ch)], m[q], sr[q])

        def wait_r(q):
            pltpu.make_async_copy(msg_hbm.at[pl.ds(0, ch)], m[q], sr[q]).wait()

        def fire_s(j, q):
            pltpu.async_copy(m[q], acc_sh.at[idx_v.at[j]], ss[q], add=True)

        def wait_s(q):
            pltpu.make_async_copy(m[q], acc_sh.at[pl.ds(0, ch)], ss[q]).wait()

        for q in range(nslot):
            fire_r(q, q)

        def body(p, carry):
            for q in range(nslot):
                j = p * nslot + q
                wait_r(q)
                fire_s(j, q)

                @pl.when(j + nslot < nchunk)
                def _(q=q, j=j):
                    wait_s(q)
                    fire_r(j + nslot, q)

            return carry

        lax.fori_loop(0, (nchunk - 1) // nslot, body, 0)
        wait_r(0)
        fire_s(nchunk - 1, 0)
        for q in range(nslot):
            wait_s(q)
        plsc.subcore_barrier()

        r0 = sid * nps
        pltpu.sync_copy(acc_sh.at[pl.ds(r0, nps)], out_hbm.at[cid, pl.ds(r0, nps)])

    return k(msg, dst4)


def _node_mlp(x, partials, w1a, w1b, b1, w2, b2, blk):
    n, d = x.shape
    nparts = len(partials)

    def body(x_ref, *refs):
        part_refs = refs[:nparts]
        wa_ref, wb_ref, b1_ref, w2_ref, b2_ref, o_ref = refs[nparts:]
        agg = part_refs[0][...]
        for pr in part_refs[1:]:
            agg = agg + pr[...]
        h = jnp.maximum(
            jnp.dot(x_ref[...], wa_ref[...], preferred_element_type=jnp.float32)
            + jnp.dot(agg, wb_ref[...], preferred_element_type=jnp.float32)
            + b1_ref[...],
            0.0,
        )
        o_ref[...] = (
            jnp.dot(h, w2_ref[...], preferred_element_type=jnp.float32)
            + b2_ref[...]
        )

    return pl.pallas_call(
        body,
        grid=(n // blk,),
        in_specs=[pl.BlockSpec((blk, d), lambda i: (i, 0))] * (1 + nparts)
        + [
            pl.BlockSpec((d, d), lambda i: (0, 0)),
            pl.BlockSpec((d, d), lambda i: (0, 0)),
            pl.BlockSpec((1, d), lambda i: (0, 0)),
            pl.BlockSpec((d, d), lambda i: (0, 0)),
            pl.BlockSpec((1, d), lambda i: (0, 0)),
        ],
        out_specs=pl.BlockSpec((blk, d), lambda i: (i, 0)),
        out_shape=jax.ShapeDtypeStruct((n, d), jnp.float32),
    )(x, *partials, w1a, w1b, b1, w2, b2)


def kernel(node_tensor, edge_idx_tensor, We1, be1, We2, be2, Wn1, bn1, Wn2,
           bn2):
    x = node_tensor
    n, d = x.shape
    e = edge_idx_tensor.shape[1]
    src = edge_idx_tensor[0]
    dst = edge_idx_tensor[1]

    a, b = _edge_premix(x, We1[:d], We1[d:], be1.reshape(1, d), 2000)

    nhalf = 2
    chg, gslot = 120, 3
    chs, sslot = 40, 4
    nw = _NC * _NS
    eh = e // nhalf
    epw = eh // nw
    epw_pad = epw
    while not (epw_pad % 8 == 0 and epw_pad % chg == 0
               and (epw_pad // chg - 1) % gslot == 0
               and epw_pad % chs == 0
               and (epw_pad // chs - 1) % sslot == 0):
        epw_pad += 8
    nps = ((n + _NS * 128 - 1) // (_NS * 128)) * 128
    npad = _NS * nps
    pad = epw_pad - epw

    partials = []
    for hh in range(nhalf):
        src_h = lax.slice_in_dim(src, hh * eh, (hh + 1) * eh)
        dst_h = lax.slice_in_dim(dst, hh * eh, (hh + 1) * eh)
        if pad:
            ar = jnp.arange(nw * pad, dtype=jnp.int32)
            ps = (ar % n).reshape(nw, pad)
            pd = (n + ar % (npad - n)).reshape(nw, pad)
            src_h = jnp.concatenate([src_h.reshape(nw, epw), ps], 1).reshape(-1)
            dst_h = jnp.concatenate([dst_h.reshape(nw, epw), pd], 1).reshape(-1)
        ga, gb = _sc_gather(a, b, src_h, dst_h, chg, gslot)
        msg = _edge_mlp(ga, gb, We2, be2.reshape(1, d), 1920)
        dst4 = dst_h.reshape(_NC, _NS, epw_pad // chs, chs)
        pp = _sc_scatter(msg, dst4, n, chs, sslot)[:, :n]
        partials.extend([pp[0], pp[1]])
    out = _node_mlp(x, partials, Wn1[:d], Wn1[d:],
                    bn1.reshape(1, d), Wn2, bn2.reshape(1, d), 2000)
    return (out, edge_idx_tensor)


# speedup vs baseline: 1.0224x; 1.0224x over previous
import functools

import jax
import jax.numpy as jnp
from jax import lax
from jax.experimental import pallas as pl
from jax.experimental.pallas import tpu as pltpu
from jax.experimental.pallas import tpu_sc as plsc

_NC = 2
_NS = 16


def _edge_premix(x, w1a, w1b, b1, blk):
    n, d = x.shape

    def body(x_ref, wa_ref, wb_ref, b1_ref, a_ref, b_ref):
        xv = x_ref[...]
        a_ref[...] = jnp.dot(xv, wa_ref[...], preferred_element_type=jnp.float32)
        b_ref[...] = (
            jnp.dot(xv, wb_ref[...], preferred_element_type=jnp.float32)
            + b1_ref[...]
        )

    return pl.pallas_call(
        body,
        grid=(n // blk,),
        in_specs=[
            pl.BlockSpec((blk, d), lambda i: (i, 0)),
            pl.BlockSpec((d, d), lambda i: (0, 0)),
            pl.BlockSpec((d, d), lambda i: (0, 0)),
            pl.BlockSpec((1, d), lambda i: (0, 0)),
        ],
        out_specs=[
            pl.BlockSpec((blk, d), lambda i: (i, 0)),
            pl.BlockSpec((blk, d), lambda i: (i, 0)),
        ],
        out_shape=[
            jax.ShapeDtypeStruct((n, d), jnp.float32),
            jax.ShapeDtypeStruct((n, d), jnp.float32),
        ],
    )(x, w1a, w1b, b1)


def _sc_gather(a, b, src, dst, ch, nslot=4):
    n, d = a.shape
    e = src.shape[0]
    nw = _NC * _NS
    epw = e // nw
    nchunk = epw // ch
    assert nchunk >= nslot
    mesh = plsc.VectorSubcoreMesh(
        core_axis_name="c", subcore_axis_name="s",
        num_cores=_NC, num_subcores=_NS,
    )

    @functools.partial(
        pl.kernel,
        mesh=mesh,
        out_type=(
            jax.ShapeDtypeStruct((e, d), jnp.float32),
            jax.ShapeDtypeStruct((e, d), jnp.float32),
        ),
        scratch_types=[pltpu.VMEM((epw,), jnp.int32)] * 2
        + [pltpu.VMEM((ch, d), jnp.float32)] * (2 * nslot)
        + [pltpu.SemaphoreType.DMA] * (2 * nslot),
    )
    def k(a_hbm, b_hbm, src_hbm, dst_hbm, ao_hbm, bo_hbm, src_v, dst_v,
          *rest):
        ra = rest[0:nslot]
        rb = rest[nslot:2 * nslot]
        sg = rest[2 * nslot:3 * nslot]
        sw = rest[3 * nslot:4 * nslot]
        wid = lax.axis_index("s") * _NC + lax.axis_index("c")
        base = wid * epw
        pltpu.sync_copy(src_hbm.at[pl.ds(base, epw)], src_v)
        pltpu.sync_copy(dst_hbm.at[pl.ds(base, epw)], dst_v)

        def fire_g(j, q):
            off = j * ch
            pltpu.async_copy(a_hbm.at[src_v.at[pl.ds(off, ch)]], ra[q], sg[q])
            pltpu.async_copy(b_hbm.at[dst_v.at[pl.ds(off, ch)]], rb[q], sg[q])

        def wait_g(q):
            pltpu.make_async_copy(a_hbm.at[pl.ds(0, ch)], ra[q], sg[q]).wait()
            pltpu.make_async_copy(b_hbm.at[pl.ds(0, ch)], rb[q], sg[q]).wait()

        def fire_w(j, q):
            off = base + j * ch
            pltpu.async_copy(ra[q], ao_hbm.at[pl.ds(off, ch)], sw[q])
            pltpu.async_copy(rb[q], bo_hbm.at[pl.ds(off, ch)], sw[q])

        def wait_w(q):
            pltpu.make_async_copy(ra[q], ao_hbm.at[pl.ds(0, ch)], sw[q]).wait()
            pltpu.make_async_copy(rb[q], bo_hbm.at[pl.ds(0, ch)], sw[q]).wait()

        for q in range(nslot):
            fire_g(q, q)

        def step(j, q):
            wait_g(q)
            fire_w(j, q)

            @pl.when(j + nslot < nchunk)
            def _():
                wait_w(q)
                fire_g(j + nslot, q)

        def body(p, carry):
            for q in range(nslot):
                step(p * nslot + q, q)
            return carry

        nb = nchunk // nslot
        lax.fori_loop(0, nb, body, 0)
        for q in range(nchunk - nb * nslot):
            step(nb * nslot + q, q)
        for q in range(nslot):
            wait_w(q)

    return k(a, b, src, dst)


def _edge_mlp(ga, gb, w2, b2, blk):
    e, d = ga.shape

    def body(ga_ref, gb_ref, w2_ref, b2_ref, o_ref):
        h = jnp.maximum(ga_ref[...] + gb_ref[...], 0.0)
        m = (
            jnp.dot(h, w2_ref[...], preferred_element_type=jnp.float32)
            + b2_ref[...]
        )
        o_ref[...] = jnp.maximum(m, 0.0)

    return pl.pallas_call(
        body,
        grid=(e // blk,),
        in_specs=[
            pl.BlockSpec((blk, d), lambda i: (i, 0)),
            pl.BlockSpec((blk, d), lambda i: (i, 0)),
            pl.BlockSpec((d, d), lambda i: (0, 0)),
            pl.BlockSpec((1, d), lambda i: (0, 0)),
        ],
        out_specs=pl.BlockSpec((blk, d), lambda i: (i, 0)),
        out_shape=jax.ShapeDtypeStruct((e, d), jnp.float32),
    )(ga, gb, w2, b2)


def _sc_scatter(msg, dst4, n, ch, nslot=4):
    e, d = msg.shape
    nchunk = dst4.shape[2]
    assert nchunk >= nslot
    nps = ((n + _NS * 128 - 1) // (_NS * 128)) * 128
    npad = _NS * nps
    mesh = plsc.VectorSubcoreMesh(
        core_axis_name="c", subcore_axis_name="s",
        num_cores=_NC, num_subcores=_NS,
    )

    @functools.partial(
        pl.kernel,
        mesh=mesh,
        out_type=jax.ShapeDtypeStruct((_NC, npad, d), jnp.float32),
        scratch_types=[pltpu.VMEM((nchunk, ch), jnp.int32)]
        + [pltpu.VMEM((ch, d), jnp.float32)] * nslot
        + [pltpu.VMEM_SHARED((npad, d), jnp.float32)]
        + [pltpu.SemaphoreType.DMA] * (2 * nslot),
    )
    def k(msg_hbm, dst_hbm, out_hbm, idx_v, *rest):
        m = rest[0:nslot]
        acc_sh = rest[nslot]
        sr = rest[nslot + 1:2 * nslot + 1]
        ss = rest[2 * nslot + 1:3 * nslot + 1]
        cid = lax.axis_index("c")
        sid = lax.axis_index("s")
        ebase = cid * (e // _NC) + sid * (e // (_NC * _NS))

        z = jnp.zeros((16,), jnp.float32)

        def zbody(r, carry):
            for q in range(d // 16):
                m[0][r, pl.ds(q * 16, 16)] = z
            return carry

        lax.fori_loop(0, ch, zbody, 0)
        for t in range(nps // ch):
            pltpu.sync_copy(m[0], acc_sh.at[pl.ds(sid * nps + t * ch, ch)])
        plsc.subcore_barrier()

        pltpu.sync_copy(dst_hbm.at[cid, sid], idx_v)

        def fire_r(j, q):
            pltpu.async_copy(msg_hbm.at[pl.ds(ebase + j * ch, ch)], m[q], sr[q])

        def wait_r(q):
            pltpu.make_async_copy(msg_hbm.at[pl.ds(0, ch)], m[q], sr[q]).wait()

        def fire_s(j, q):
            pltpu.async_copy(m[q], acc_sh.at[idx_v.at[j]], ss[q], add=True)

        def wait_s(q):
            pltpu.make_async_copy(m[q], acc_sh.at[pl.ds(0, ch)], ss[q]).wait()

        for q in range(nslot):
            fire_r(q, q)

        def step(j, q):
            wait_r(q)
            fire_s(j, q)

            @pl.when(j + nslot < nchunk)
            def _():
                wait_s(q)
                fire_r(j + nslot, q)

        def body(p, carry):
            for q in range(nslot):
                step(p * nslot + q, q)
            return carry

        nb = nchunk // nslot
        lax.fori_loop(0, nb, body, 0)
        for q in range(nchunk - nb * nslot):
            step(nb * nslot + q, q)
        for q in range(nslot):
            wait_s(q)
        plsc.subcore_barrier()

        r0 = sid * nps
        pltpu.sync_copy(acc_sh.at[pl.ds(r0, nps)], out_hbm.at[cid, pl.ds(r0, nps)])

    return k(msg, dst4)


def _node_mlp(x, partials, w1a, w1b, b1, w2, b2, blk):
    n, d = x.shape
    nparts = len(partials)

    def body(x_ref, *refs):
        part_refs = refs[:nparts]
        wa_ref, wb_ref, b1_ref, w2_ref, b2_ref, o_ref = refs[nparts:]
        agg = part_refs[0][...]
        for pr in part_refs[1:]:
            agg = agg + pr[...]
        h = jnp.maximum(
            jnp.dot(x_ref[...], wa_ref[...], preferred_element_type=jnp.float32)
            + jnp.dot(agg, wb_ref[...], preferred_element_type=jnp.float32)
            + b1_ref[...],
            0.0,
        )
        o_ref[...] = (
            jnp.dot(h, w2_ref[...], preferred_element_type=jnp.float32)
            + b2_ref[...]
        )

    return pl.pallas_call(
        body,
        grid=(n // blk,),
        in_specs=[pl.BlockSpec((blk, d), lambda i: (i, 0))] * (1 + nparts)
        + [
            pl.BlockSpec((d, d), lambda i: (0, 0)),
            pl.BlockSpec((d, d), lambda i: (0, 0)),
            pl.BlockSpec((1, d), lambda i: (0, 0)),
            pl.BlockSpec((d, d), lambda i: (0, 0)),
            pl.BlockSpec((1, d), lambda i: (0, 0)),
        ],
        out_specs=pl.BlockSpec((blk, d), lambda i: (i, 0)),
        out_shape=jax.ShapeDtypeStruct((n, d), jnp.float32),
    )(x, *partials, w1a, w1b, b1, w2, b2)


def kernel(node_tensor, edge_idx_tensor, We1, be1, We2, be2, Wn1, bn1, Wn2,
           bn2):
    x = node_tensor
    n, d = x.shape
    e = edge_idx_tensor.shape[1]
    src = edge_idx_tensor[0]
    dst = edge_idx_tensor[1]

    a, b = _edge_premix(x, We1[:d], We1[d:], be1.reshape(1, d), 2000)

    nhalf = 2
    chg, gslot = 80, 4
    chs, sslot = 40, 4
    nw = _NC * _NS
    eh = e // nhalf
    epw = eh // nw
    epw_pad = epw
    while not (epw_pad % 8 == 0 and epw_pad % chg == 0
               and epw_pad % chs == 0):
        epw_pad += 8
    nps = ((n + _NS * 128 - 1) // (_NS * 128)) * 128
    npad = _NS * nps
    pad = epw_pad - epw

    partials = []
    for hh in range(nhalf):
        src_h = lax.slice_in_dim(src, hh * eh, (hh + 1) * eh)
        dst_h = lax.slice_in_dim(dst, hh * eh, (hh + 1) * eh)
        if pad:
            ar = jnp.arange(nw * pad, dtype=jnp.int32)
            ps = (ar % n).reshape(nw, pad)
            pd = (n + ar % (npad - n)).reshape(nw, pad)
            src_h = jnp.concatenate([src_h.reshape(nw, epw), ps], 1).reshape(-1)
            dst_h = jnp.concatenate([dst_h.reshape(nw, epw), pd], 1).reshape(-1)
        ga, gb = _sc_gather(a, b, src_h, dst_h, chg, gslot)
        msg = _edge_mlp(ga, gb, We2, be2.reshape(1, d), 1920)
        dst4 = dst_h.reshape(_NC, _NS, epw_pad // chs, chs)
        pp = _sc_scatter(msg, dst4, n, chs, sslot)[:, :n]
        partials.extend([pp[0], pp[1]])
    out = _node_mlp(x, partials, Wn1[:d], Wn1[d:],
                    bn1.reshape(1, d), Wn2, bn2.reshape(1, d), 2000)
    return (out, edge_idx_tensor)


# speedup vs baseline: 1.0483x; 1.0254x over previous
import functools

import jax
import jax.numpy as jnp
from jax import lax
from jax.experimental import pallas as pl
from jax.experimental.pallas import tpu as pltpu
from jax.experimental.pallas import tpu_sc as plsc

_NC = 2
_NS = 16


def _edge_premix(x, w1a, w1b, b1, blk):
    n, d = x.shape

    def body(x_ref, wa_ref, wb_ref, b1_ref, a_ref, b_ref):
        xv = x_ref[...]
        a_ref[...] = jnp.dot(xv, wa_ref[...], preferred_element_type=jnp.float32)
        b_ref[...] = (
            jnp.dot(xv, wb_ref[...], preferred_element_type=jnp.float32)
            + b1_ref[...]
        )

    return pl.pallas_call(
        body,
        grid=(n // blk,),
        in_specs=[
            pl.BlockSpec((blk, d), lambda i: (i, 0)),
            pl.BlockSpec((d, d), lambda i: (0, 0)),
            pl.BlockSpec((d, d), lambda i: (0, 0)),
            pl.BlockSpec((1, d), lambda i: (0, 0)),
        ],
        out_specs=[
            pl.BlockSpec((blk, d), lambda i: (i, 0)),
            pl.BlockSpec((blk, d), lambda i: (i, 0)),
        ],
        out_shape=[
            jax.ShapeDtypeStruct((n, d), jnp.float32),
            jax.ShapeDtypeStruct((n, d), jnp.float32),
        ],
    )(x, w1a, w1b, b1)


def _sc_gather(a, b, src, dst, ch, nslot=4):
    n, d = a.shape
    e = src.shape[0]
    nw = _NC * _NS
    epw = e // nw
    nchunk = epw // ch
    assert nchunk >= nslot
    mesh = plsc.VectorSubcoreMesh(
        core_axis_name="c", subcore_axis_name="s",
        num_cores=_NC, num_subcores=_NS,
    )

    @functools.partial(
        pl.kernel,
        mesh=mesh,
        out_type=(
            jax.ShapeDtypeStruct((e, d), jnp.float32),
            jax.ShapeDtypeStruct((e, d), jnp.float32),
        ),
        scratch_types=[pltpu.VMEM((epw,), jnp.int32)] * 2
        + [pltpu.VMEM((ch, d), jnp.float32)] * (2 * nslot)
        + [pltpu.SemaphoreType.DMA] * (2 * nslot),
    )
    def k(a_hbm, b_hbm, src_hbm, dst_hbm, ao_hbm, bo_hbm, src_v, dst_v,
          *rest):
        ra = rest[0:nslot]
        rb = rest[nslot:2 * nslot]
        sg = rest[2 * nslot:3 * nslot]
        sw = rest[3 * nslot:4 * nslot]
        wid = lax.axis_index("s") * _NC + lax.axis_index("c")
        base = wid * epw
        pltpu.sync_copy(src_hbm.at[pl.ds(base, epw)], src_v)
        pltpu.sync_copy(dst_hbm.at[pl.ds(base, epw)], dst_v)

        def fire_g(j, q):
            off = j * ch
            pltpu.async_copy(a_hbm.at[src_v.at[pl.ds(off, ch)]], ra[q], sg[q])
            pltpu.async_copy(b_hbm.at[dst_v.at[pl.ds(off, ch)]], rb[q], sg[q])

        def wait_g(q):
            pltpu.make_async_copy(a_hbm.at[pl.ds(0, ch)], ra[q], sg[q]).wait()
            pltpu.make_async_copy(b_hbm.at[pl.ds(0, ch)], rb[q], sg[q]).wait()

        def fire_w(j, q):
            off = base + j * ch
            pltpu.async_copy(ra[q], ao_hbm.at[pl.ds(off, ch)], sw[q])
            pltpu.async_copy(rb[q], bo_hbm.at[pl.ds(off, ch)], sw[q])

        def wait_w(q):
            pltpu.make_async_copy(ra[q], ao_hbm.at[pl.ds(0, ch)], sw[q]).wait()
            pltpu.make_async_copy(rb[q], bo_hbm.at[pl.ds(0, ch)], sw[q]).wait()

        for q in range(nslot):
            fire_g(q, q)

        def step(j, q):
            wait_g(q)
            fire_w(j, q)

            @pl.when(j + nslot < nchunk)
            def _():
                wait_w(q)
                fire_g(j + nslot, q)

        def body(p, carry):
            for q in range(nslot):
                step(p * nslot + q, q)
            return carry

        nb = nchunk // nslot
        lax.fori_loop(0, nb, body, 0)
        for q in range(nchunk - nb * nslot):
            step(nb * nslot + q, q)
        for q in range(nslot):
            wait_w(q)

    return k(a, b, src, dst)


def _edge_mlp(ga, gb, w2, b2, blk):
    e, d = ga.shape

    def body(ga_ref, gb_ref, w2_ref, b2_ref, o_ref):
        h = jnp.maximum(ga_ref[...] + gb_ref[...], 0.0)
        m = (
            jnp.dot(h, w2_ref[...], preferred_element_type=jnp.float32)
            + b2_ref[...]
        )
        o_ref[...] = jnp.maximum(m, 0.0)

    return pl.pallas_call(
        body,
        grid=(e // blk,),
        in_specs=[
            pl.BlockSpec((blk, d), lambda i: (i, 0)),
            pl.BlockSpec((blk, d), lambda i: (i, 0)),
            pl.BlockSpec((d, d), lambda i: (0, 0)),
            pl.BlockSpec((1, d), lambda i: (0, 0)),
        ],
        out_specs=pl.BlockSpec((blk, d), lambda i: (i, 0)),
        out_shape=jax.ShapeDtypeStruct((e, d), jnp.float32),
    )(ga, gb, w2, b2)


def _sc_scatter(msg, dst4, n, ch, nslot=4):
    e, d = msg.shape
    nchunk = dst4.shape[2]
    assert nchunk >= nslot
    nps = ((n + _NS * 128 - 1) // (_NS * 128)) * 128
    npad = _NS * nps
    mesh = plsc.VectorSubcoreMesh(
        core_axis_name="c", subcore_axis_name="s",
        num_cores=_NC, num_subcores=_NS,
    )

    @functools.partial(
        pl.kernel,
        mesh=mesh,
        out_type=jax.ShapeDtypeStruct((_NC, npad, d), jnp.float32),
        scratch_types=[pltpu.VMEM((nchunk, ch), jnp.int32)]
        + [pltpu.VMEM((ch, d), jnp.float32)] * nslot
        + [pltpu.VMEM_SHARED((npad, d), jnp.float32)]
        + [pltpu.SemaphoreType.DMA] * (2 * nslot),
    )
    def k(msg_hbm, dst_hbm, out_hbm, idx_v, *rest):
        m = rest[0:nslot]
        acc_sh = rest[nslot]
        sr = rest[nslot + 1:2 * nslot + 1]
        ss = rest[2 * nslot + 1:3 * nslot + 1]
        cid = lax.axis_index("c")
        sid = lax.axis_index("s")
        ebase = cid * (e // _NC) + sid * (e // (_NC * _NS))

        z = jnp.zeros((16,), jnp.float32)

        def zbody(r, carry):
            for q in range(d // 16):
                m[0][r, pl.ds(q * 16, 16)] = z
            return carry

        lax.fori_loop(0, ch, zbody, 0)
        for t in range(nps // ch):
            pltpu.sync_copy(m[0], acc_sh.at[pl.ds(sid * nps + t * ch, ch)])
        plsc.subcore_barrier()

        pltpu.sync_copy(dst_hbm.at[cid, sid], idx_v)

        def fire_r(j, q):
            pltpu.async_copy(msg_hbm.at[pl.ds(ebase + j * ch, ch)], m[q], sr[q])

        def wait_r(q):
            pltpu.make_async_copy(msg_hbm.at[pl.ds(0, ch)], m[q], sr[q]).wait()

        def fire_s(j, q):
            pltpu.async_copy(m[q], acc_sh.at[idx_v.at[j]], ss[q], add=True)

        def wait_s(q):
            pltpu.make_async_copy(m[q], acc_sh.at[pl.ds(0, ch)], ss[q]).wait()

        for q in range(nslot):
            fire_r(q, q)

        def step(j, q):
            wait_r(q)
            fire_s(j, q)

            @pl.when(j + nslot < nchunk)
            def _():
                wait_s(q)
                fire_r(j + nslot, q)

        def body(p, carry):
            for q in range(nslot):
                step(p * nslot + q, q)
            return carry

        nb = nchunk // nslot
        lax.fori_loop(0, nb, body, 0)
        for q in range(nchunk - nb * nslot):
            step(nb * nslot + q, q)
        for q in range(nslot):
            wait_s(q)
        plsc.subcore_barrier()

        r0 = sid * nps
        pltpu.sync_copy(acc_sh.at[pl.ds(r0, nps)], out_hbm.at[cid, pl.ds(r0, nps)])

    return k(msg, dst4)


def _node_mlp(x, partials, w1a, w1b, b1, w2, b2, blk):
    n, d = x.shape
    nparts = len(partials)

    def body(x_ref, *refs):
        part_refs = refs[:nparts]
        wa_ref, wb_ref, b1_ref, w2_ref, b2_ref, o_ref = refs[nparts:]
        agg = part_refs[0][...]
        for pr in part_refs[1:]:
            agg = agg + pr[...]
        h = jnp.maximum(
            jnp.dot(x_ref[...], wa_ref[...], preferred_element_type=jnp.float32)
            + jnp.dot(agg, wb_ref[...], preferred_element_type=jnp.float32)
            + b1_ref[...],
            0.0,
        )
        o_ref[...] = (
            jnp.dot(h, w2_ref[...], preferred_element_type=jnp.float32)
            + b2_ref[...]
        )

    return pl.pallas_call(
        body,
        grid=(n // blk,),
        in_specs=[pl.BlockSpec((blk, d), lambda i: (i, 0))] * (1 + nparts)
        + [
            pl.BlockSpec((d, d), lambda i: (0, 0)),
            pl.BlockSpec((d, d), lambda i: (0, 0)),
            pl.BlockSpec((1, d), lambda i: (0, 0)),
            pl.BlockSpec((d, d), lambda i: (0, 0)),
            pl.BlockSpec((1, d), lambda i: (0, 0)),
        ],
        out_specs=pl.BlockSpec((blk, d), lambda i: (i, 0)),
        out_shape=jax.ShapeDtypeStruct((n, d), jnp.float32),
    )(x, *partials, w1a, w1b, b1, w2, b2)


def kernel(node_tensor, edge_idx_tensor, We1, be1, We2, be2, Wn1, bn1, Wn2,
           bn2):
    x = node_tensor
    n, d = x.shape
    e = edge_idx_tensor.shape[1]
    src = edge_idx_tensor[0]
    dst = edge_idx_tensor[1]

    a, b = _edge_premix(x, We1[:d], We1[d:], be1.reshape(1, d), 2000)

    nhalf = 2
    chg, gslot = 40, 4
    chs, sslot = 40, 4
    nw = _NC * _NS
    eh = e // nhalf
    epw = eh // nw
    epw_pad = epw
    while not (epw_pad % 8 == 0 and epw_pad % chg == 0
               and epw_pad % chs == 0):
        epw_pad += 8
    nps = ((n + _NS * 128 - 1) // (_NS * 128)) * 128
    npad = _NS * nps
    pad = epw_pad - epw

    partials = []
    for hh in range(nhalf):
        src_h = lax.slice_in_dim(src, hh * eh, (hh + 1) * eh)
        dst_h = lax.slice_in_dim(dst, hh * eh, (hh + 1) * eh)
        if pad:
            ar = jnp.arange(nw * pad, dtype=jnp.int32)
            ps = (ar % n).reshape(nw, pad)
            pd = (n + ar % (npad - n)).reshape(nw, pad)
            src_h = jnp.concatenate([src_h.reshape(nw, epw), ps], 1).reshape(-1)
            dst_h = jnp.concatenate([dst_h.reshape(nw, epw), pd], 1).reshape(-1)
        ga, gb = _sc_gather(a, b, src_h, dst_h, chg, gslot)
        msg = _edge_mlp(ga, gb, We2, be2.reshape(1, d), 1920)
        dst4 = dst_h.reshape(_NC, _NS, epw_pad // chs, chs)
        pp = _sc_scatter(msg, dst4, n, chs, sslot)[:, :n]
        partials.extend([pp[0], pp[1]])
    out = _node_mlp(x, partials, Wn1[:d], Wn1[d:],
                    bn1.reshape(1, d), Wn2, bn2.reshape(1, d), 2000)
    return (out, edge_idx_tensor)


# speedup vs baseline: 1.0525x; 1.0040x over previous
import functools

import jax
import jax.numpy as jnp
from jax import lax
from jax.experimental import pallas as pl
from jax.experimental.pallas import tpu as pltpu
from jax.experimental.pallas import tpu_sc as plsc

_NC = 2
_NS = 16


def _edge_premix(x, w1a, w1b, b1, blk):
    n, d = x.shape

    def body(x_ref, wa_ref, wb_ref, b1_ref, a_ref, b_ref):
        xv = x_ref[...]
        a_ref[...] = jnp.dot(xv, wa_ref[...], preferred_element_type=jnp.float32)
        b_ref[...] = (
            jnp.dot(xv, wb_ref[...], preferred_element_type=jnp.float32)
            + b1_ref[...]
        )

    return pl.pallas_call(
        body,
        grid=(n // blk,),
        in_specs=[
            pl.BlockSpec((blk, d), lambda i: (i, 0)),
            pl.BlockSpec((d, d), lambda i: (0, 0)),
            pl.BlockSpec((d, d), lambda i: (0, 0)),
            pl.BlockSpec((1, d), lambda i: (0, 0)),
        ],
        out_specs=[
            pl.BlockSpec((blk, d), lambda i: (i, 0)),
            pl.BlockSpec((blk, d), lambda i: (i, 0)),
        ],
        out_shape=[
            jax.ShapeDtypeStruct((n, d), jnp.float32),
            jax.ShapeDtypeStruct((n, d), jnp.float32),
        ],
    )(x, w1a, w1b, b1)


def _sc_gather(a, b, src, dst, ch, nslot=4):
    n, d = a.shape
    e = src.shape[0]
    nw = _NC * _NS
    epw = e // nw
    nchunk = epw // ch
    assert nchunk >= nslot
    mesh = plsc.VectorSubcoreMesh(
        core_axis_name="c", subcore_axis_name="s",
        num_cores=_NC, num_subcores=_NS,
    )

    @functools.partial(
        pl.kernel,
        mesh=mesh,
        out_type=(
            jax.ShapeDtypeStruct((e, d), jnp.float32),
            jax.ShapeDtypeStruct((e, d), jnp.float32),
        ),
        scratch_types=[pltpu.VMEM((epw,), jnp.int32)] * 2
        + [pltpu.VMEM((ch, d), jnp.float32)] * (2 * nslot)
        + [pltpu.SemaphoreType.DMA] * (4 * nslot),
    )
    def k(a_hbm, b_hbm, src_hbm, dst_hbm, ao_hbm, bo_hbm, src_v, dst_v,
          *rest):
        ra = rest[0:nslot]
        rb = rest[nslot:2 * nslot]
        sga = rest[2 * nslot:3 * nslot]
        sgb = rest[3 * nslot:4 * nslot]
        swa = rest[4 * nslot:5 * nslot]
        swb = rest[5 * nslot:6 * nslot]
        wid = lax.axis_index("s") * _NC + lax.axis_index("c")
        base = wid * epw
        pltpu.sync_copy(src_hbm.at[pl.ds(base, epw)], src_v)
        pltpu.sync_copy(dst_hbm.at[pl.ds(base, epw)], dst_v)

        def fire_g(j, q):
            off = j * ch
            pltpu.async_copy(a_hbm.at[src_v.at[pl.ds(off, ch)]], ra[q], sga[q])
            pltpu.async_copy(b_hbm.at[dst_v.at[pl.ds(off, ch)]], rb[q], sgb[q])

        def wait_g(q):
            pltpu.make_async_copy(a_hbm.at[pl.ds(0, ch)], ra[q], sga[q]).wait()
            pltpu.make_async_copy(b_hbm.at[pl.ds(0, ch)], rb[q], sgb[q]).wait()

        def fire_w(j, q):
            off = base + j * ch
            pltpu.async_copy(ra[q], ao_hbm.at[pl.ds(off, ch)], swa[q])
            pltpu.async_copy(rb[q], bo_hbm.at[pl.ds(off, ch)], swb[q])

        def wait_w(q):
            pltpu.make_async_copy(ra[q], ao_hbm.at[pl.ds(0, ch)], swa[q]).wait()
            pltpu.make_async_copy(rb[q], bo_hbm.at[pl.ds(0, ch)], swb[q]).wait()

        for q in range(nslot):
            fire_g(q, q)

        def step(j, q):
            wait_g(q)
            fire_w(j, q)

            @pl.when(j + nslot < nchunk)
            def _():
                wait_w(q)
                fire_g(j + nslot, q)

        def body(p, carry):
            for q in range(nslot):
                step(p * nslot + q, q)
            return carry

        nb = nchunk // nslot
        lax.fori_loop(0, nb, body, 0)
        for q in range(nchunk - nb * nslot):
            step(nb * nslot + q, q)
        for q in range(nslot):
            wait_w(q)

    return k(a, b, src, dst)


def _edge_mlp(ga, gb, w2, b2):
    e, d = ga.shape
    blk = 2048
    while e % blk or blk % 8:
        blk -= 8

    def body(ga_ref, gb_ref, w2_ref, b2_ref, o_ref):
        h = jnp.maximum(ga_ref[...] + gb_ref[...], 0.0)
        m = (
            jnp.dot(h, w2_ref[...], preferred_element_type=jnp.float32)
            + b2_ref[...]
        )
        o_ref[...] = jnp.maximum(m, 0.0)

    return pl.pallas_call(
        body,
        grid=(e // blk,),
        in_specs=[
            pl.BlockSpec((blk, d), lambda i: (i, 0)),
            pl.BlockSpec((blk, d), lambda i: (i, 0)),
            pl.BlockSpec((d, d), lambda i: (0, 0)),
            pl.BlockSpec((1, d), lambda i: (0, 0)),
        ],
        out_specs=pl.BlockSpec((blk, d), lambda i: (i, 0)),
        out_shape=jax.ShapeDtypeStruct((e, d), jnp.float32),
    )(ga, gb, w2, b2)


def _sc_scatter(msg, dst4, n, ch, nslot=4):
    e, d = msg.shape
    nchunk = dst4.shape[2]
    assert nchunk >= nslot
    nps = ((n + _NS * 128 - 1) // (_NS * 128)) * 128
    npad = _NS * nps
    mesh = plsc.VectorSubcoreMesh(
        core_axis_name="c", subcore_axis_name="s",
        num_cores=_NC, num_subcores=_NS,
    )

    @functools.partial(
        pl.kernel,
        mesh=mesh,
        out_type=jax.ShapeDtypeStruct((_NC, npad, d), jnp.float32),
        scratch_types=[pltpu.VMEM((nchunk, ch), jnp.int32)]
        + [pltpu.VMEM((ch, d), jnp.float32)] * nslot
        + [pltpu.VMEM_SHARED((npad, d), jnp.float32)]
        + [pltpu.SemaphoreType.DMA] * (2 * nslot),
    )
    def k(msg_hbm, dst_hbm, out_hbm, idx_v, *rest):
        m = rest[0:nslot]
        acc_sh = rest[nslot]
        sr = rest[nslot + 1:2 * nslot + 1]
        ss = rest[2 * nslot + 1:3 * nslot + 1]
        cid = lax.axis_index("c")
        sid = lax.axis_index("s")
        ebase = cid * (e // _NC) + sid * (e // (_NC * _NS))

        z = jnp.zeros((16,), jnp.float32)

        def zbody(r, carry):
            for q in range(d // 16):
                m[0][r, pl.ds(q * 16, 16)] = z
            return carry

        lax.fori_loop(0, ch, zbody, 0)
        for t in range(nps // ch):
            pltpu.sync_copy(m[0], acc_sh.at[pl.ds(sid * nps + t * ch, ch)])
        plsc.subcore_barrier()

        pltpu.sync_copy(dst_hbm.at[cid, sid], idx_v)

        def fire_r(j, q):
            pltpu.async_copy(msg_hbm.at[pl.ds(ebase + j * ch, ch)], m[q], sr[q])

        def wait_r(q):
            pltpu.make_async_copy(msg_hbm.at[pl.ds(0, ch)], m[q], sr[q]).wait()

        def fire_s(j, q):
            pltpu.async_copy(m[q], acc_sh.at[idx_v.at[j]], ss[q], add=True)

        def wait_s(q):
            pltpu.make_async_copy(m[q], acc_sh.at[pl.ds(0, ch)], ss[q]).wait()

        for q in range(nslot):
            fire_r(q, q)

        def step(j, q):
            wait_r(q)
            fire_s(j, q)

            @pl.when(j + nslot < nchunk)
            def _():
                wait_s(q)
                fire_r(j + nslot, q)

        def body(p, carry):
            for q in range(nslot):
                step(p * nslot + q, q)
            return carry

        nb = nchunk // nslot
        lax.fori_loop(0, nb, body, 0)
        for q in range(nchunk - nb * nslot):
            step(nb * nslot + q, q)
        for q in range(nslot):
            wait_s(q)
        plsc.subcore_barrier()

        r0 = sid * nps
        pltpu.sync_copy(acc_sh.at[pl.ds(r0, nps)], out_hbm.at[cid, pl.ds(r0, nps)])

    return k(msg, dst4)


def _node_mlp(x, partials, w1a, w1b, b1, w2, b2, blk):
    n, d = x.shape
    nparts = len(partials)

    def body(x_ref, *refs):
        part_refs = refs[:nparts]
        wa_ref, wb_ref, b1_ref, w2_ref, b2_ref, o_ref = refs[nparts:]
        agg = part_refs[0][...]
        for pr in part_refs[1:]:
            agg = agg + pr[...]
        h = jnp.maximum(
            jnp.dot(x_ref[...], wa_ref[...], preferred_element_type=jnp.float32)
            + jnp.dot(agg, wb_ref[...], preferred_element_type=jnp.float32)
            + b1_ref[...],
            0.0,
        )
        o_ref[...] = (
            jnp.dot(h, w2_ref[...], preferred_element_type=jnp.float32)
            + b2_ref[...]
        )

    return pl.pallas_call(
        body,
        grid=(n // blk,),
        in_specs=[pl.BlockSpec((blk, d), lambda i: (i, 0))] * (1 + nparts)
        + [
            pl.BlockSpec((d, d), lambda i: (0, 0)),
            pl.BlockSpec((d, d), lambda i: (0, 0)),
            pl.BlockSpec((1, d), lambda i: (0, 0)),
            pl.BlockSpec((d, d), lambda i: (0, 0)),
            pl.BlockSpec((1, d), lambda i: (0, 0)),
        ],
        out_specs=pl.BlockSpec((blk, d), lambda i: (i, 0)),
        out_shape=jax.ShapeDtypeStruct((n, d), jnp.float32),
    )(x, *partials, w1a, w1b, b1, w2, b2)


def kernel(node_tensor, edge_idx_tensor, We1, be1, We2, be2, Wn1, bn1, Wn2,
           bn2):
    x = node_tensor
    n, d = x.shape
    e = edge_idx_tensor.shape[1]
    src = edge_idx_tensor[0]
    dst = edge_idx_tensor[1]

    a, b = _edge_premix(x, We1[:d], We1[d:], be1.reshape(1, d), 2000)

    nhalf = 2
    chg, gslot = 40, 4
    chs, sslot = 40, 4
    nw = _NC * _NS
    eh = e // nhalf
    epw = eh // nw
    epw_pad = epw
    while not (epw_pad % 8 == 0 and epw_pad % chg == 0
               and epw_pad % chs == 0):
        epw_pad += 8
    nps = ((n + _NS * 128 - 1) // (_NS * 128)) * 128
    npad = _NS * nps
    pad = epw_pad - epw

    partials = []
    for hh in range(nhalf):
        src_h = lax.slice_in_dim(src, hh * eh, (hh + 1) * eh)
        dst_h = lax.slice_in_dim(dst, hh * eh, (hh + 1) * eh)
        if pad:
            ar = jnp.arange(nw * pad, dtype=jnp.int32)
            ps = (ar % n).reshape(nw, pad)
            pd = (n + ar % (npad - n)).reshape(nw, pad)
            src_h = jnp.concatenate([src_h.reshape(nw, epw), ps], 1).reshape(-1)
            dst_h = jnp.concatenate([dst_h.reshape(nw, epw), pd], 1).reshape(-1)
        ga, gb = _sc_gather(a, b, src_h, dst_h, chg, gslot)
        msg = _edge_mlp(ga, gb, We2, be2.reshape(1, d))
        dst4 = dst_h.reshape(_NC, _NS, epw_pad // chs, chs)
        pp = _sc_scatter(msg, dst4, n, chs, sslot)[:, :n]
        partials.extend([pp[0], pp[1]])
    out = _node_mlp(x, partials, Wn1[:d], Wn1[d:],
                    bn1.reshape(1, d), Wn2, bn2.reshape(1, d), 2000)
    return (out, edge_idx_tensor)
